# baseline (speedup 1.0000x reference)
import functools

import jax
import jax.numpy as jnp
from jax.experimental import pallas as pl

EPS = 1e-5


def _dense1_body(agg_ref, nd_ref, w1_ref, b1_ref, g1_ref, be1_ref, al1_ref,
                 w2_ref, z_ref):
    a = agg_ref[...] * nd_ref[...]
    h = jnp.dot(a, w1_ref[...], preferred_element_type=jnp.float32) + b1_ref[...]
    mean = jnp.mean(h, axis=0, keepdims=True)
    xc = h - al1_ref[...] * mean
    var = jnp.mean(xc * xc, axis=0, keepdims=True)
    h = g1_ref[...] * xc * jax.lax.rsqrt(var + EPS) + be1_ref[...]
    h = jnp.maximum(h, 0.0)
    z_ref[...] = jnp.dot(h, w2_ref[...], preferred_element_type=jnp.float32)


def _dense2_body(acc_ref, nd_ref, b2_ref, g2_ref, be2_ref, al2_ref, out_ref):
    v = nd_ref[...] * acc_ref[...] + b2_ref[...]
    mean = jnp.mean(v, axis=0, keepdims=True)
    xc = v - al2_ref[...] * mean
    var = jnp.mean(xc * xc, axis=0, keepdims=True)
    out_ref[...] = jax.nn.sigmoid(
        g2_ref[...] * xc * jax.lax.rsqrt(var + EPS) + be2_ref[...])


def kernel(inputs, edges, edges_weight, W1, b1, gn1_gamma, gn1_beta, gn1_alpha,
           W2, b2, gn2_gamma, gn2_beta, gn2_alpha):
    x = inputs
    n, f = x.shape
    h = W1.shape[1]
    src = edges[0].astype(jnp.int32)
    dst = edges[1].astype(jnp.int32)
    ew = edges_weight

    deg_out = jnp.zeros((n,), jnp.float32).at[src].add(ew)
    deg_in = jnp.zeros((n,), jnp.float32).at[dst].add(ew)
    norm_src = jnp.where(deg_out > 0, jax.lax.rsqrt(deg_out), 0.0)
    norm_dst = jnp.where(deg_in > 0, jax.lax.rsqrt(deg_in), 0.0)
    coef = ew * norm_src[src]

    agg = jnp.zeros((n, f), jnp.float32).at[dst].add(x[src] * coef[:, None])

    z = pl.pallas_call(
        _dense1_body,
        out_shape=jax.ShapeDtypeStruct((n, 1), jnp.float32),
    )(agg, norm_dst[:, None], W1, b1[None, :], gn1_gamma[None, :],
      gn1_beta[None, :], gn1_alpha[None, :], W2)

    acc2 = jnp.zeros((n,), jnp.float32).at[dst].add(z[:, 0][src] * coef)

    out = pl.pallas_call(
        _dense2_body,
        out_shape=jax.ShapeDtypeStruct((n, 1), jnp.float32),
    )(acc2[:, None], norm_dst[:, None], b2[None, :], gn2_gamma[None, :],
      gn2_beta[None, :], gn2_alpha[None, :])
    return out


# speedup vs baseline: 1.0028x; 1.0028x over previous
import functools

import jax
import jax.numpy as jnp
from jax.experimental import pallas as pl

EPS = 1e-5


def _dense1_body(agg_ref, nd_ref, w1_ref, b1_ref, g1_ref, be1_ref, al1_ref,
                 w2_ref, z_ref):
    a = agg_ref[...] * nd_ref[...]
    h = jnp.dot(a, w1_ref[...], preferred_element_type=jnp.float32) + b1_ref[...]
    mean = jnp.mean(h, axis=0, keepdims=True)
    xc = h - al1_ref[...] * mean
    var = jnp.mean(xc * xc, axis=0, keepdims=True)
    h = g1_ref[...] * xc * jax.lax.rsqrt(var + EPS) + be1_ref[...]
    h = jnp.maximum(h, 0.0)
    z_ref[...] = jnp.dot(h, w2_ref[...], preferred_element_type=jnp.float32)


def _dense2_body(acc_ref, nd_ref, b2_ref, g2_ref, be2_ref, al2_ref, out_ref):
    v = nd_ref[...] * acc_ref[...] + b2_ref[...]
    mean = jnp.mean(v, axis=0, keepdims=True)
    xc = v - al2_ref[...] * mean
    var = jnp.mean(xc * xc, axis=0, keepdims=True)
    out_ref[...] = jax.nn.sigmoid(
        g2_ref[...] * xc * jax.lax.rsqrt(var + EPS) + be2_ref[...])


def kernel(inputs, edges, edges_weight, W1, b1, gn1_gamma, gn1_beta, gn1_alpha,
           W2, b2, gn2_gamma, gn2_beta, gn2_alpha):
    x = inputs
    n, f = x.shape
    h = W1.shape[1]
    src = edges[0].astype(jnp.int32)
    dst = edges[1].astype(jnp.int32)
    ew = edges_weight

    deg_out = jnp.zeros((n,), jnp.float32).at[src].add(ew)
    deg_in = jnp.zeros((n,), jnp.float32).at[dst].add(ew)
    norm_src = jnp.where(deg_out > 0, jax.lax.rsqrt(deg_out), 0.0)
    norm_dst = jnp.where(deg_in > 0, jax.lax.rsqrt(deg_in), 0.0)
    coef = ew * norm_src[src]

    agg = jnp.zeros((n, f), jnp.float32).at[dst].add(x[src] * coef[:, None])

    a = agg * norm_dst[:, None]
    hh = a @ W1 + b1
    mean = jnp.mean(hh, 0, keepdims=True)
    xc = hh - gn1_alpha * mean
    var = jnp.mean(xc * xc, 0, keepdims=True)
    hh = jnp.maximum(gn1_gamma * xc * jax.lax.rsqrt(var + EPS) + gn1_beta, 0.0)
    z = hh @ W2

    acc2 = jnp.zeros((n,), jnp.float32).at[dst].add(z[:, 0][src] * coef)

    v = norm_dst[:, None] * acc2[:, None] + b2
    mean = jnp.mean(v, 0, keepdims=True)
    xc = v - gn2_alpha * mean
    var = jnp.mean(xc * xc, 0, keepdims=True)
    out = jax.nn.sigmoid(gn2_gamma * xc * jax.lax.rsqrt(var + EPS) + gn2_beta)
    return out


# speedup vs baseline: 15.5132x; 15.4706x over previous
import functools

import jax
import jax.numpy as jnp
from jax import lax
from jax.experimental import pallas as pl
from jax.experimental.pallas import tpu as pltpu
from jax.experimental.pallas import tpu_sc as plsc

EPS = 1e-5
NC = 2
NS = 16
L = 16


def _rsqrt16(x):
    i = plsc.bitcast(x, jnp.int32)
    y = plsc.bitcast(jnp.int32(0x5F3759DF) - (i >> 1), jnp.float32)
    for _ in range(3):
        y = y * (1.5 - 0.5 * x * y * y)
    return jnp.where(x > 0, y, 0.0)


def _zero_ref(ref, nwords):
    zero16 = jnp.zeros((L,), jnp.float32)

    def zb(i, _):
        ref[pl.ds(i * L, L)] = zero16
        return 0

    lax.fori_loop(0, nwords // L, zb, 0)


def _make_deg_norm(e, npad, ept, slc):
    mesh = plsc.VectorSubcoreMesh(core_axis_name="c", subcore_axis_name="s", num_cores=NC, num_subcores=NS)

    @functools.partial(
        pl.kernel,
        mesh=mesh,
        compiler_params=pltpu.CompilerParams(needs_layout_passes=False),
        out_type=[
            jax.ShapeDtypeStruct((npad,), jnp.float32),
            jax.ShapeDtypeStruct((npad,), jnp.float32),
        ],
        scratch_types=[
            pltpu.VMEM((ept,), jnp.int32),
            pltpu.VMEM((ept,), jnp.int32),
            pltpu.VMEM((ept,), jnp.float32),
            pltpu.VMEM((npad,), jnp.float32),
            pltpu.VMEM((npad,), jnp.float32),
            pltpu.VMEM_SHARED((NS * npad,), jnp.float32),
            pltpu.VMEM_SHARED((NS * npad,), jnp.float32),
            pltpu.VMEM((slc,), jnp.float32),
            pltpu.VMEM((slc,), jnp.float32),
            pltpu.VMEM((slc,), jnp.float32),
            pltpu.VMEM((slc,), jnp.float32),
        ],
    )
    def deg_norm(src_h, dst_h, ew_h, nsrc_h, ndst_h, srcb, dstb, ewb,
                 dego, degi, sho, shi, tmpo, tmpi, acco, acci):
        c = lax.axis_index("c")
        s = lax.axis_index("s")
        base = s * ept
        pltpu.sync_copy(src_h.at[pl.ds(base, ept)], srcb)
        pltpu.sync_copy(dst_h.at[pl.ds(base, ept)], dstb)
        pltpu.sync_copy(ew_h.at[pl.ds(base, ept)], ewb)
        _zero_ref(dego, npad)
        _zero_ref(degi, npad)

        def eb(i, _):
            o = i * L
            sv = srcb[pl.ds(o, L)]
            dv = dstb[pl.ds(o, L)]
            ev = ewb[pl.ds(o, L)]
            plsc.addupdate_scatter(dego, [sv], ev)
            plsc.addupdate_scatter(degi, [dv], ev)
            return 0

        lax.fori_loop(0, ept // L, eb, 0)
        pltpu.sync_copy(dego, sho.at[pl.ds(s * npad, npad)])
        pltpu.sync_copy(degi, shi.at[pl.ds(s * npad, npad)])
        plsc.subcore_barrier()
        rbase = s * slc
        pltpu.sync_copy(sho.at[pl.ds(rbase, slc)], acco)
        pltpu.sync_copy(shi.at[pl.ds(rbase, slc)], acci)

        def rb(p, _):
            pltpu.sync_copy(sho.at[pl.ds(p * npad + rbase, slc)], tmpo)
            pltpu.sync_copy(shi.at[pl.ds(p * npad + rbase, slc)], tmpi)

            def ab(j, _):
                sl = pl.ds(j * L, L)
                acco[sl] = acco[sl] + tmpo[sl]
                acci[sl] = acci[sl] + tmpi[sl]
                return 0

            lax.fori_loop(0, slc // L, ab, 0)
            return 0

        lax.fori_loop(1, NS, rb, 0)

        def nb(j, _):
            sl = pl.ds(j * L, L)
            acco[sl] = _rsqrt16(acco[sl])
            acci[sl] = _rsqrt16(acci[sl])
            return 0

        lax.fori_loop(0, slc // L, nb, 0)

        @pl.when(c == 0)
        def _():
            pltpu.sync_copy(acco, nsrc_h.at[pl.ds(rbase, slc)])

        @pl.when(c == 1)
        def _():
            pltpu.sync_copy(acci, ndst_h.at[pl.ds(rbase, slc)])

    return deg_norm


def _make_agg1(n, f, e, npad, ept, ch, slc):
    mesh = plsc.VectorSubcoreMesh(core_axis_name="c", subcore_axis_name="s", num_cores=NC, num_subcores=NS)
    nch = ept // ch

    @functools.partial(
        pl.kernel,
        mesh=mesh,
        compiler_params=pltpu.CompilerParams(needs_layout_passes=False),
        out_type=jax.ShapeDtypeStruct((NC * npad, f), jnp.float32),
        scratch_types=[
            pltpu.VMEM((npad,), jnp.float32),
            pltpu.VMEM((ept,), jnp.int32),
            pltpu.VMEM((ept,), jnp.float32),
            pltpu.VMEM((ch,), jnp.int32),
            pltpu.VMEM((ch + L,), jnp.float32),
            pltpu.VMEM((ch, f), jnp.float32),
            pltpu.VMEM_SHARED((npad, f), jnp.float32),
        ],
    )
    def agg1(x_h, src_h, dst_h, ew_h, ns_h, agg_h,
             nsb, srcb, ewb, dstc, coefb, rows, sh_agg):
        c = lax.axis_index("c")
        s = lax.axis_index("s")
        w = s * NC + c
        ebase = w * ept
        pltpu.sync_copy(ns_h, nsb)
        pltpu.sync_copy(src_h.at[pl.ds(ebase, ept)], srcb)
        pltpu.sync_copy(ew_h.at[pl.ds(ebase, ept)], ewb)
        zero16 = jnp.zeros((L,), jnp.float32)

        def zrow(i, _):
            for fb in range(f // L):
                rows[i, pl.ds(fb * L, L)] = zero16
            return 0

        lax.fori_loop(0, ch, zrow, 0)
        rbase = s * slc
        for k in range(slc // ch):
            pltpu.sync_copy(rows, sh_agg.at[pl.ds(rbase + k * ch, ch)])
        plsc.subcore_barrier()

        def chunk(i, _):
            eoff = i * ch

            def cg(g, _):
                o = eoff + g * L
                sv = srcb[pl.ds(o, L)]
                nsv = plsc.load_gather(nsb, [sv])
                coefb[pl.ds(g * L, L)] = ewb[pl.ds(o, L)] * nsv
                return 0

            lax.fori_loop(0, ch // L, cg, 0)
            pltpu.sync_copy(dst_h.at[pl.ds(ebase + eoff, ch)], dstc)
            pltpu.sync_copy(x_h.at[srcb.at[pl.ds(eoff, ch)]], rows)

            def eb(e2, _):
                cs = coefb[pl.ds(e2, L)][0]
                for fb in range(f // L):
                    sl = pl.ds(fb * L, L)
                    rows[e2, sl] = rows[e2, sl] * cs
                return 0

            lax.fori_loop(0, ch, eb, 0)
            pltpu.sync_copy(rows, sh_agg.at[dstc], add=True)
            return 0

        lax.fori_loop(0, nch, chunk, 0)
        plsc.subcore_barrier()
        pltpu.sync_copy(sh_agg.at[pl.ds(rbase, slc)],
                        agg_h.at[pl.ds(c * npad + rbase, slc)])

    return agg1


def _make_agg2(e, npad, ept, slc):
    mesh = plsc.VectorSubcoreMesh(core_axis_name="c", subcore_axis_name="s", num_cores=NC, num_subcores=NS)

    @functools.partial(
        pl.kernel,
        mesh=mesh,
        compiler_params=pltpu.CompilerParams(needs_layout_passes=False),
        out_type=jax.ShapeDtypeStruct((NC * npad,), jnp.float32),
        scratch_types=[
            pltpu.VMEM((npad,), jnp.float32),
            pltpu.VMEM((npad,), jnp.float32),
            pltpu.VMEM((ept,), jnp.int32),
            pltpu.VMEM((ept,), jnp.int32),
            pltpu.VMEM((ept,), jnp.float32),
            pltpu.VMEM((npad,), jnp.float32),
            pltpu.VMEM_SHARED((NS * npad,), jnp.float32),
            pltpu.VMEM((slc,), jnp.float32),
            pltpu.VMEM((slc,), jnp.float32),
        ],
    )
    def agg2(z_h, src_h, dst_h, ew_h, ns_h, out_h,
             zb, nsb, srcb, dstb, ewb, accb, shp, tmp, accs):
        c = lax.axis_index("c")
        s = lax.axis_index("s")
        w = s * NC + c
        ebase = w * ept
        pltpu.sync_copy(z_h, zb)
        pltpu.sync_copy(ns_h, nsb)
        pltpu.sync_copy(src_h.at[pl.ds(ebase, ept)], srcb)
        pltpu.sync_copy(dst_h.at[pl.ds(ebase, ept)], dstb)
        pltpu.sync_copy(ew_h.at[pl.ds(ebase, ept)], ewb)
        _zero_ref(accb, npad)

        def eb(i, _):
            o = i * L
            sv = srcb[pl.ds(o, L)]
            dv = dstb[pl.ds(o, L)]
            ev = ewb[pl.ds(o, L)]
            zg = plsc.load_gather(zb, [sv])
            ns = plsc.load_gather(nsb, [sv])
            plsc.addupdate_scatter(accb, [dv], zg * ev * ns)
            return 0

        lax.fori_loop(0, ept // L, eb, 0)
        pltpu.sync_copy(accb, shp.at[pl.ds(s * npad, npad)])
        plsc.subcore_barrier()
        rbase = s * slc
        pltpu.sync_copy(shp.at[pl.ds(rbase, slc)], accs)

        def rb(p, _):
            pltpu.sync_copy(shp.at[pl.ds(p * npad + rbase, slc)], tmp)

            def ab(j, _):
                sl = pl.ds(j * L, L)
                accs[sl] = accs[sl] + tmp[sl]
                return 0

            lax.fori_loop(0, slc // L, ab, 0)
            return 0

        lax.fori_loop(1, NS, rb, 0)
        pltpu.sync_copy(accs, out_h.at[pl.ds(c * npad + rbase, slc)])

    return agg2


def _dense1_body(n, aggp_ref, nd_ref, mask_ref, w1_ref, b1_ref, g1_ref,
                 be1_ref, al1_ref, w2_ref, z_ref):
    a = (aggp_ref[0] + aggp_ref[1]) * nd_ref[...]
    h = jnp.dot(a, w1_ref[...], preferred_element_type=jnp.float32)
    h = (h + b1_ref[...]) * mask_ref[...]
    s1 = jnp.sum(h, axis=0, keepdims=True)
    s2 = jnp.sum(h * h, axis=0, keepdims=True)
    al = al1_ref[...]
    mean = s1 * (1.0 / n)
    var = s2 * (1.0 / n) - (2.0 * al) * mean * (s1 * (1.0 / n)) \
        + al * al * mean * mean
    hn = g1_ref[...] * (h - al * mean) * lax.rsqrt(var + EPS) + be1_ref[...]
    hn = jnp.maximum(hn, 0.0)
    z_ref[...] = jnp.dot(hn, w2_ref[...], preferred_element_type=jnp.float32)


def _dense2_body(n, accp_ref, nd_ref, mask_ref, b2_ref, g2_ref, be2_ref,
                 al2_ref, out_ref):
    v = (accp_ref[0] + accp_ref[1]) * nd_ref[...]
    v = (v + b2_ref[0, 0]) * mask_ref[...]
    s1 = jnp.sum(v)
    s2 = jnp.sum(v * v)
    al = al2_ref[0, 0]
    mean = s1 * (1.0 / n)
    var = s2 * (1.0 / n) - (2.0 * al) * mean * (s1 * (1.0 / n)) \
        + al * al * mean * mean
    out_ref[...] = jax.nn.sigmoid(
        g2_ref[0, 0] * (v - al * mean) * lax.rsqrt(var + EPS) + be2_ref[0, 0])


def kernel(inputs, edges, edges_weight, W1, b1, gn1_gamma, gn1_beta, gn1_alpha,
           W2, b2, gn2_gamma, gn2_beta, gn2_alpha):
    x = inputs
    n, f = x.shape
    e = edges.shape[1]
    slc = ((n + NS * L - 1) // (NS * L)) * L
    npad = slc * NS
    assert e % (NC * NS * L) == 0
    src = edges[0].astype(jnp.int32)
    dst = edges[1].astype(jnp.int32)
    ew = edges_weight

    nsrc, ndst = _make_deg_norm(e, npad, e // NS, slc)(src, dst, ew)

    aggp = _make_agg1(n, f, e, npad, e // (NC * NS), 80, slc)(
        x, src, dst, ew, nsrc).reshape(NC, npad, f)

    mask = (jnp.arange(npad) < n).astype(jnp.float32)[:, None]
    z = pl.pallas_call(
        functools.partial(_dense1_body, n),
        out_shape=jax.ShapeDtypeStruct((npad, 1), jnp.float32),
    )(aggp, ndst[:, None], mask, W1, b1[None, :], gn1_gamma[None, :],
      gn1_beta[None, :], gn1_alpha[None, :], W2)

    acc2p = _make_agg2(e, npad, e // (NC * NS), slc)(
        z.reshape(npad), src, dst, ew, nsrc).reshape(NC, npad)

    rows2 = npad // 128
    out = pl.pallas_call(
        functools.partial(_dense2_body, n),
        out_shape=jax.ShapeDtypeStruct((rows2, 128), jnp.float32),
    )(acc2p.reshape(NC, rows2, 128), ndst.reshape(rows2, 128),
      mask.reshape(rows2, 128), b2[None, :], gn2_gamma[None, :],
      gn2_beta[None, :], gn2_alpha[None, :])
    return out.reshape(npad)[:n, None]


# speedup vs baseline: 27.6013x; 1.7792x over previous
import functools

import jax
import jax.numpy as jnp
from jax import lax
from jax.experimental import pallas as pl
from jax.experimental.pallas import tpu as pltpu
from jax.experimental.pallas import tpu_sc as plsc

EPS = 1e-5
NC = 2
NS = 16
L = 16


def _rsqrt16(x):
    i = plsc.bitcast(x, jnp.int32)
    y = plsc.bitcast(jnp.int32(0x5F3759DF) - (i >> 1), jnp.float32)
    for _ in range(3):
        y = y * (1.5 - 0.5 * x * y * y)
    return jnp.where(x > 0, y, 0.0)


def _zero_ref(ref, nwords):
    zero16 = jnp.zeros((L,), jnp.float32)

    def zb(i, _):
        ref[pl.ds(i * L, L)] = zero16
        return 0

    lax.fori_loop(0, nwords // L, zb, 0)


def _make_deg(e, npad, ept):
    mesh = plsc.VectorSubcoreMesh(core_axis_name="c", subcore_axis_name="s", num_cores=NC, num_subcores=NS)

    @functools.partial(
        pl.kernel,
        mesh=mesh,
        compiler_params=pltpu.CompilerParams(needs_layout_passes=False),
        out_type=[
            jax.ShapeDtypeStruct((NC * NS * npad,), jnp.float32),
            jax.ShapeDtypeStruct((NC * NS * npad,), jnp.float32),
        ],
        scratch_types=[
            pltpu.VMEM((ept,), jnp.int32),
            pltpu.VMEM((ept,), jnp.int32),
            pltpu.VMEM((ept,), jnp.float32),
            pltpu.VMEM((npad,), jnp.float32),
            pltpu.VMEM((npad,), jnp.float32),
        ],
    )
    def deg(src_h, dst_h, ew_h, dego_h, degi_h, srcb, dstb, ewb, dego, degi):
        c = lax.axis_index("c")
        s = lax.axis_index("s")
        w = s * NC + c
        base = w * ept
        pltpu.sync_copy(src_h.at[pl.ds(base, ept)], srcb)
        pltpu.sync_copy(dst_h.at[pl.ds(base, ept)], dstb)
        pltpu.sync_copy(ew_h.at[pl.ds(base, ept)], ewb)
        _zero_ref(dego, npad)
        _zero_ref(degi, npad)

        def eb(i, _):
            o = i * L
            sv = srcb[pl.ds(o, L)]
            dv = dstb[pl.ds(o, L)]
            ev = ewb[pl.ds(o, L)]
            plsc.addupdate_scatter(dego, [sv], ev)
            plsc.addupdate_scatter(degi, [dv], ev)
            return 0

        lax.fori_loop(0, ept // L, eb, 0, unroll=4)
        pltpu.sync_copy(dego, dego_h.at[pl.ds(w * npad, npad)])
        pltpu.sync_copy(degi, degi_h.at[pl.ds(w * npad, npad)])

    return deg


def _norms_body(degp_o_ref, degp_i_ref, nsrc_ref, ndst_ref):
    do = jnp.sum(degp_o_ref[...], axis=0)
    di = jnp.sum(degp_i_ref[...], axis=0)
    nsrc_ref[...] = jnp.where(do > 0, lax.rsqrt(do), 0.0)
    ndst_ref[...] = jnp.where(di > 0, lax.rsqrt(di), 0.0)


def _xs_body(x_ref, ns_ref, xs_ref):
    xs_ref[...] = x_ref[...] * ns_ref[...]


def _make_agg1(n, f, e, npad, ept, ch, slc):
    mesh = plsc.VectorSubcoreMesh(core_axis_name="c", subcore_axis_name="s", num_cores=NC, num_subcores=NS)
    nch = ept // ch
    assert nch % 2 == 1 and ept % ch == 0 and slc % ch == 0

    @functools.partial(
        pl.kernel,
        mesh=mesh,
        compiler_params=pltpu.CompilerParams(needs_layout_passes=False),
        out_type=jax.ShapeDtypeStruct((NC * npad, f), jnp.float32),
        scratch_types=[
            pltpu.VMEM((ept,), jnp.int32),
            pltpu.VMEM((ept + L,), jnp.float32),
            pltpu.VMEM((ch,), jnp.int32),
            pltpu.VMEM((ch,), jnp.int32),
            pltpu.VMEM((ch, f), jnp.float32),
            pltpu.VMEM((ch, f), jnp.float32),
            pltpu.VMEM_SHARED((npad, f), jnp.float32),
            pltpu.SemaphoreType.DMA,
            pltpu.SemaphoreType.DMA,
            pltpu.SemaphoreType.DMA,
            pltpu.SemaphoreType.DMA,
        ],
    )
    def agg1(x_h, src_h, dst_h, ew_h, agg_h,
             srcb, ewb, dstc0, dstc1, rows0, rows1, sh_agg,
             gsem0, gsem1, dsem0, dsem1):
        c = lax.axis_index("c")
        s = lax.axis_index("s")
        w = s * NC + c
        ebase = w * ept
        pltpu.sync_copy(src_h.at[pl.ds(ebase, ept)], srcb)
        pltpu.sync_copy(ew_h.at[pl.ds(ebase, ept)], ewb.at[pl.ds(0, ept)])
        zero16 = jnp.zeros((L,), jnp.float32)

        def zrow(i, _):
            for fb in range(f // L):
                rows0[i, pl.ds(fb * L, L)] = zero16
            return 0

        lax.fori_loop(0, ch, zrow, 0)
        rbase = s * slc
        for k in range(slc // ch):
            pltpu.sync_copy(rows0, sh_agg.at[pl.ds(rbase + k * ch, ch)])

        plsc.subcore_barrier()

        def issue(cc, dstc, rows, gsem, dsem):
            pltpu.async_copy(dst_h.at[pl.ds(ebase + cc * ch, ch)], dstc, dsem)
            pltpu.async_copy(x_h.at[srcb.at[pl.ds(cc * ch, ch)]], rows, gsem)

        def wait(cc, dstc, rows, gsem, dsem):
            pltpu.make_async_copy(
                dst_h.at[pl.ds(ebase + cc * ch, ch)], dstc, dsem).wait()
            pltpu.make_async_copy(
                x_h.at[srcb.at[pl.ds(cc * ch, ch)]], rows, gsem).wait()

        def scale(cc, rows):
            cbase = cc * ch

            def eb(e2, _):
                cs = ewb[pl.ds(cbase + e2, L)][0]
                for fb in range(f // L):
                    sl = pl.ds(fb * L, L)
                    rows[e2, sl] = rows[e2, sl] * cs
                return 0

            lax.fori_loop(0, ch, eb, 0, unroll=4)

        issue(0, dstc0, rows0, gsem0, dsem0)
        issue(1, dstc1, rows1, gsem1, dsem1)

        def pair(i, _):
            a = 2 * i
            wait(a, dstc0, rows0, gsem0, dsem0)
            scale(a, rows0)
            pltpu.sync_copy(rows0, sh_agg.at[dstc0], add=True)

            @pl.when(a + 2 < nch)
            def _():
                issue(a + 2, dstc0, rows0, gsem0, dsem0)

            b = a + 1
            wait(b, dstc1, rows1, gsem1, dsem1)
            scale(b, rows1)
            pltpu.sync_copy(rows1, sh_agg.at[dstc1], add=True)

            @pl.when(b + 2 < nch)
            def _():
                issue(b + 2, dstc1, rows1, gsem1, dsem1)

            return 0

        lax.fori_loop(0, nch // 2, pair, 0)
        last = nch - 1
        wait(last, dstc0, rows0, gsem0, dsem0)
        scale(last, rows0)
        pltpu.sync_copy(rows0, sh_agg.at[dstc0], add=True)
        plsc.subcore_barrier()
        pltpu.sync_copy(sh_agg.at[pl.ds(rbase, slc)],
                        agg_h.at[pl.ds(c * npad + rbase, slc)])

    return agg1


def _make_agg2(e, npad, ept):
    mesh = plsc.VectorSubcoreMesh(core_axis_name="c", subcore_axis_name="s", num_cores=NC, num_subcores=NS)

    @functools.partial(
        pl.kernel,
        mesh=mesh,
        compiler_params=pltpu.CompilerParams(needs_layout_passes=False),
        out_type=jax.ShapeDtypeStruct((NC * NS * npad,), jnp.float32),
        scratch_types=[
            pltpu.VMEM((npad,), jnp.float32),
            pltpu.VMEM((ept,), jnp.int32),
            pltpu.VMEM((ept,), jnp.int32),
            pltpu.VMEM((ept,), jnp.float32),
            pltpu.VMEM((npad,), jnp.float32),
        ],
    )
    def agg2(z_h, src_h, dst_h, ew_h, out_h,
             zb, srcb, dstb, ewb, accb):
        c = lax.axis_index("c")
        s = lax.axis_index("s")
        w = s * NC + c
        ebase = w * ept
        pltpu.sync_copy(z_h, zb)
        pltpu.sync_copy(src_h.at[pl.ds(ebase, ept)], srcb)
        pltpu.sync_copy(dst_h.at[pl.ds(ebase, ept)], dstb)
        pltpu.sync_copy(ew_h.at[pl.ds(ebase, ept)], ewb)
        _zero_ref(accb, npad)

        def eb(i, _):
            o = i * L
            sv = srcb[pl.ds(o, L)]
            dv = dstb[pl.ds(o, L)]
            ev = ewb[pl.ds(o, L)]
            zg = plsc.load_gather(zb, [sv])
            plsc.addupdate_scatter(accb, [dv], zg * ev)
            return 0

        lax.fori_loop(0, ept // L, eb, 0, unroll=4)
        pltpu.sync_copy(accb, out_h.at[pl.ds(w * npad, npad)])

    return agg2


def _dense1_body(n, aggp_ref, nd_ref, ns_ref, mask_ref, w1_ref, b1_ref,
                 g1_ref, be1_ref, al1_ref, w2_ref, z_ref):
    a = (aggp_ref[0] + aggp_ref[1]) * nd_ref[...]
    h = jnp.dot(a, w1_ref[...], preferred_element_type=jnp.float32)
    h = (h + b1_ref[...]) * mask_ref[...]
    s1 = jnp.sum(h, axis=0, keepdims=True)
    s2 = jnp.sum(h * h, axis=0, keepdims=True)
    al = al1_ref[...]
    mean = s1 * (1.0 / n)
    var = s2 * (1.0 / n) - (2.0 * al) * mean * (s1 * (1.0 / n)) \
        + al * al * mean * mean
    hn = g1_ref[...] * (h - al * mean) * lax.rsqrt(var + EPS) + be1_ref[...]
    hn = jnp.maximum(hn, 0.0)
    z_ref[...] = jnp.dot(
        hn, w2_ref[...], preferred_element_type=jnp.float32) * ns_ref[...]


def _dense2_body(n, accp_ref, nd_ref, mask_ref, b2_ref, g2_ref, be2_ref,
                 al2_ref, out_ref):
    v = jnp.sum(accp_ref[...], axis=0) * nd_ref[...]
    v = (v + b2_ref[0, 0]) * mask_ref[...]
    s1 = jnp.sum(v)
    s2 = jnp.sum(v * v)
    al = al2_ref[0, 0]
    mean = s1 * (1.0 / n)
    var = s2 * (1.0 / n) - (2.0 * al) * mean * (s1 * (1.0 / n)) \
        + al * al * mean * mean
    out_ref[...] = jax.nn.sigmoid(
        g2_ref[0, 0] * (v - al * mean) * lax.rsqrt(var + EPS) + be2_ref[0, 0])


def kernel(inputs, edges, edges_weight, W1, b1, gn1_gamma, gn1_beta, gn1_alpha,
           W2, b2, gn2_gamma, gn2_beta, gn2_alpha):
    x = inputs
    n, f = x.shape
    e = edges.shape[1]
    slc = ((n + NS * L - 1) // (NS * L)) * L
    npad = slc * NS
    assert e % (NC * NS * L) == 0
    src = edges[0].astype(jnp.int32)
    dst = edges[1].astype(jnp.int32)
    ew = edges_weight

    rows2 = npad // 128
    degp_o, degp_i = _make_deg(e, npad, e // (NC * NS))(src, dst, ew)
    nsrc2d, ndst2d = pl.pallas_call(
        _norms_body,
        out_shape=[
            jax.ShapeDtypeStruct((rows2, 128), jnp.float32),
            jax.ShapeDtypeStruct((rows2, 128), jnp.float32),
        ],
    )(degp_o.reshape(NC * NS, rows2, 128), degp_i.reshape(NC * NS, rows2, 128))
    ndst = ndst2d.reshape(npad)
    nscol = nsrc2d.reshape(npad)[:, None]
    xs = pl.pallas_call(
        _xs_body,
        out_shape=jax.ShapeDtypeStruct((n, f), jnp.float32),
    )(x, nscol[:n])

    aggp = _make_agg1(n, f, e, npad, e // (NC * NS), 80, slc)(
        xs, src, dst, ew).reshape(NC, npad, f)

    mask = (jnp.arange(npad) < n).astype(jnp.float32)[:, None]
    z = pl.pallas_call(
        functools.partial(_dense1_body, n),
        out_shape=jax.ShapeDtypeStruct((npad, 1), jnp.float32),
    )(aggp, ndst[:, None], nscol, mask, W1, b1[None, :], gn1_gamma[None, :],
      gn1_beta[None, :], gn1_alpha[None, :], W2)

    acc2p = _make_agg2(e, npad, e // (NC * NS))(
        z.reshape(npad), src, dst, ew)

    out = pl.pallas_call(
        functools.partial(_dense2_body, n),
        out_shape=jax.ShapeDtypeStruct((rows2, 128), jnp.float32),
    )(acc2p.reshape(NC * NS, rows2, 128), ndst.reshape(rows2, 128),
      mask.reshape(rows2, 128), b2[None, :], gn2_gamma[None, :],
      gn2_beta[None, :], gn2_alpha[None, :])
    return out.reshape(npad)[:n, None]
